# Initial kernel scaffold; baseline (speedup 1.0000x reference)
import jax
import jax.numpy as jnp
from jax.experimental import pallas as pl


def kernel(nodes, edges, globals_, senders, receivers, n_node, n_edge, edge_weights, W_node, b_node, W_edge, b_edge, W_gn, b_gn, W_ge, b_ge, W_g, b_g, W_fg, b_fg):
    raise NotImplementedError("write your pallas kernel here")


# speedup vs baseline: 1.9868x; 1.9868x over previous
import functools

import jax
import jax.numpy as jnp
from jax import lax
from jax.experimental import pallas as pl
from jax.experimental.pallas import tpu as pltpu
from jax.experimental.pallas import tpu_sc as plsc

F32 = jnp.float32
I32 = jnp.int32

N = 10000
D = 128
E = 320000
DE = 16
DG = 16
ZW = 160

NC = 2
NS = 16
NW = NC * NS
EPW = E // NW
C = 80
NCH = EPW // C
RPT = N // NS
L = 16


def _pre_body(nodes_ref, wcat_ref, g_ref, weg_ref, wg4_ref, wg_ref, bg_ref,
              z_ref, gev_ref, gn_ref, tg_ref):
    z = jnp.dot(nodes_ref[...], wcat_ref[...], preferred_element_type=F32)
    z_ref[...] = z
    g = g_ref[...]
    gev_ref[...] = jnp.dot(g, weg_ref[...], preferred_element_type=F32)
    gn_ref[...] = jnp.dot(g, wg4_ref[...], preferred_element_type=F32)
    tg_ref[...] = jnp.dot(g, wg_ref[...], preferred_element_type=F32) + bg_ref[...]


def _pre(nodes, wcat, g, weg, wg4, wg, bg):
    nb = 10
    return pl.pallas_call(
        _pre_body,
        grid=(nb,),
        in_specs=[
            pl.BlockSpec((N // nb, D), lambda i: (i, 0)),
            pl.BlockSpec((D, ZW), lambda i: (0, 0)),
            pl.BlockSpec((1, DG), lambda i: (0, 0)),
            pl.BlockSpec((DG, DE), lambda i: (0, 0)),
            pl.BlockSpec((DG, D), lambda i: (0, 0)),
            pl.BlockSpec((DG, DG), lambda i: (0, 0)),
            pl.BlockSpec((1, DG), lambda i: (0, 0)),
        ],
        out_specs=[
            pl.BlockSpec((N // nb, ZW), lambda i: (i, 0)),
            pl.BlockSpec((1, DE), lambda i: (0, 0)),
            pl.BlockSpec((1, D), lambda i: (0, 0)),
            pl.BlockSpec((1, DG), lambda i: (0, 0)),
        ],
        out_shape=[
            jax.ShapeDtypeStruct((N, ZW), F32),
            jax.ShapeDtypeStruct((1, DE), F32),
            jax.ShapeDtypeStruct((1, D), F32),
            jax.ShapeDtypeStruct((1, DG), F32),
        ],
    )(nodes, wcat, g, weg, wg4, wg, bg)


def _sc_body(z_hbm, qrt_hbm, s_hbm, r_hbm, ew_hbm, ed_hbm, gev_hbm,
             acc_hbm, u_hbm,
             V, si, ri, ewv, edv, qrv, ub, gevv, acc_sh):
    cid = lax.axis_index("c")
    sid = lax.axis_index("s")
    wid = cid * NS + sid

    pltpu.sync_copy(gev_hbm, gevv)
    zeros16 = jnp.zeros((L,), F32)

    def zrow(i, _):
        for j in range(ZW // L):
            V[i, j * L:(j + 1) * L] = zeros16
        return 0

    lax.fori_loop(0, C, zrow, 0)

    def zchunk(t, _):
        k = sid + t * NS

        @pl.when(k * C < N)
        def _():
            pltpu.sync_copy(V, acc_sh.at[pl.ds(k * C, C)])
        return 0

    lax.fori_loop(0, pl.cdiv(N, C * NS), zchunk, 0)
    plsc.subcore_barrier()

    iota = lax.iota(I32, L)
    m0 = iota == 0
    onehot1 = jnp.where(iota == 1, jnp.ones((L,), F32), zeros16)
    gvec = gevv[...]

    def chunk(ch, _):
        base = wid * EPW + ch * C
        pltpu.sync_copy(s_hbm.at[pl.ds(base, C)], si)
        pltpu.sync_copy(r_hbm.at[pl.ds(base, C)], ri)
        pltpu.sync_copy(ew_hbm.at[pl.ds(base, C)], ewv)
        pltpu.sync_copy(ed_hbm.at[pl.ds(base, C)], edv)
        pltpu.sync_copy(z_hbm.at[si], V)
        pltpu.sync_copy(qrt_hbm.at[ri], qrv)

        def edge(e, _):
            w = plsc.load_gather(ewv, [jnp.full((L,), e, I32)])
            qs = V[e, 8 * L:9 * L]
            ub[e, :] = (qs + qrv[e, :] + gvec) * w
            for j in range(8):
                V[e, j * L:(j + 1) * L] = V[e, j * L:(j + 1) * L] * w
            V[e, 8 * L:9 * L] = edv[e, :] * w
            V[e, 9 * L:10 * L] = jnp.where(m0, w, onehot1)
            return 0

        lax.fori_loop(0, C, edge, 0)
        pltpu.sync_copy(V, acc_sh.at[ri], add=True)
        pltpu.sync_copy(ub, u_hbm.at[pl.ds(base, C)])
        return 0

    lax.fori_loop(0, NCH, chunk, 0)

    plsc.subcore_barrier()

    def fchunk(t, _):
        k = sid + t * NS

        @pl.when(k * C < N)
        def _():
            pltpu.sync_copy(acc_sh.at[pl.ds(k * C, C)],
                            acc_hbm.at[cid, pl.ds(k * C, C)])
        return 0

    lax.fori_loop(0, pl.cdiv(N, C * NS), fchunk, 0)


def _sc_run(z, qrt, senders, receivers, ew, edges, gev):
    mesh = plsc.VectorSubcoreMesh(core_axis_name="c", subcore_axis_name="s",
                                  num_cores=NC, num_subcores=NS)
    f = pl.kernel(
        _sc_body,
        out_type=[
            jax.ShapeDtypeStruct((NC, N, ZW), F32),
            jax.ShapeDtypeStruct((E, DE), F32),
        ],
        mesh=mesh,
        compiler_params=pltpu.CompilerParams(needs_layout_passes=False,
                                             use_tc_tiling_on_sc=False),
        scratch_types=[
            pltpu.VMEM((C, ZW), F32),
            pltpu.VMEM((C,), I32),
            pltpu.VMEM((C,), I32),
            pltpu.VMEM((C,), F32),
            pltpu.VMEM((C, DE), F32),
            pltpu.VMEM((C, DE), F32),
            pltpu.VMEM((C, DE), F32),
            pltpu.VMEM((DE,), F32),
            pltpu.VMEM_SHARED((N, ZW), F32),
        ],
    )
    return f(z, qrt, senders, receivers, ew, edges, gev)


def _edge_post_body(u_ref, ew_ref, ed_ref, wee_ref, be_ref, out_ref):
    ef = ew_ref[...] * ed_ref[...]
    out_ref[...] = (u_ref[...] + jnp.dot(ef, wee_ref[...], preferred_element_type=F32)
                    + be_ref[...])


def _edge_post(u, ew2, edges, wee, be):
    eb = 80
    blk = E // eb
    return pl.pallas_call(
        _edge_post_body,
        grid=(eb,),
        in_specs=[
            pl.BlockSpec((blk, DE), lambda i: (i, 0)),
            pl.BlockSpec((blk, 1), lambda i: (i, 0)),
            pl.BlockSpec((blk, DE), lambda i: (i, 0)),
            pl.BlockSpec((DE, DE), lambda i: (0, 0)),
            pl.BlockSpec((1, DE), lambda i: (0, 0)),
        ],
        out_specs=pl.BlockSpec((blk, DE), lambda i: (i, 0)),
        out_shape=jax.ShapeDtypeStruct((E, DE), F32),
    )(u, ew2, edges, wee, be)


def _node_post_body(acc_ref, nodes_ref, we_ref, wr_ref, gn_ref, bn_ref,
                    tg_ref, wgn_ref, bgn_ref, wge_ref, bge_ref, wfg_ref, bfg_ref,
                    nn_ref, ng_ref, nsum, esum):
    i = pl.program_id(0)
    a = acc_ref[0] + acc_ref[1]
    a0 = a[:, 0:D]
    ae = a[:, D:D + DE]
    sw = a[:, D + DE:D + DE + 1]
    cnt = a[:, D + DE + 1:D + DE + 2]
    nodes = nodes_ref[...]
    nn = (a0 + jnp.dot(ae, we_ref[...], preferred_element_type=F32)
          + jnp.dot(sw * nodes, wr_ref[...], preferred_element_type=F32)
          + sw * gn_ref[...] + cnt * bn_ref[...])
    nn_ref[...] = nn

    @pl.when(i == 0)
    def _():
        nsum[...] = jnp.zeros_like(nsum)
        esum[...] = jnp.zeros_like(esum)

    nsum[...] += jnp.sum(nodes, axis=0, keepdims=True)
    esum[...] += jnp.sum(ae, axis=0, keepdims=True)

    @pl.when(i == pl.num_programs(0) - 1)
    def _():
        tng = jnp.dot(nsum[...], wgn_ref[...], preferred_element_type=F32) + bgn_ref[...]
        teg = jnp.dot(esum[...], wge_ref[...], preferred_element_type=F32) + bge_ref[...]
        ng_ref[...] = (jnp.dot(tg_ref[...], wfg_ref[0:DG], preferred_element_type=F32)
                       + jnp.dot(tng, wfg_ref[DG:2 * DG], preferred_element_type=F32)
                       + jnp.dot(teg, wfg_ref[2 * DG:3 * DG], preferred_element_type=F32)
                       + bfg_ref[...])


def _node_post(acc, nodes, we, wr, gn, bn2, tg, wgn, bgn2, wge, bge2, wfg, bfg2):
    nb = 10
    blk = N // nb
    return pl.pallas_call(
        _node_post_body,
        grid=(nb,),
        in_specs=[
            pl.BlockSpec((NC, blk, ZW), lambda i: (0, i, 0)),
            pl.BlockSpec((blk, D), lambda i: (i, 0)),
            pl.BlockSpec((DE, D), lambda i: (0, 0)),
            pl.BlockSpec((D, D), lambda i: (0, 0)),
            pl.BlockSpec((1, D), lambda i: (0, 0)),
            pl.BlockSpec((1, D), lambda i: (0, 0)),
            pl.BlockSpec((1, DG), lambda i: (0, 0)),
            pl.BlockSpec((D, DG), lambda i: (0, 0)),
            pl.BlockSpec((1, DG), lambda i: (0, 0)),
            pl.BlockSpec((DE, DG), lambda i: (0, 0)),
            pl.BlockSpec((1, DG), lambda i: (0, 0)),
            pl.BlockSpec((3 * DG, DG), lambda i: (0, 0)),
            pl.BlockSpec((1, DG), lambda i: (0, 0)),
        ],
        out_specs=[
            pl.BlockSpec((blk, D), lambda i: (i, 0)),
            pl.BlockSpec((1, DG), lambda i: (0, 0)),
        ],
        out_shape=[
            jax.ShapeDtypeStruct((N, D), F32),
            jax.ShapeDtypeStruct((1, DG), F32),
        ],
        scratch_shapes=[
            pltpu.VMEM((1, D), F32),
            pltpu.VMEM((1, DE), F32),
        ],
    )(acc, nodes, we, wr, gn, bn2, tg, wgn, bgn2, wge, bge2, wfg, bfg2)


def kernel(nodes, edges, globals_, senders, receivers, n_node, n_edge,
           edge_weights, W_node, b_node, W_edge, b_edge, W_gn, b_gn,
           W_ge, b_ge, W_g, b_g, W_fg, b_fg):
    W_s, W_r = W_node[:D], W_node[D:2 * D]
    W_e, W_g4 = W_node[2 * D:2 * D + DE], W_node[2 * D + DE:]
    W_es, W_er = W_edge[:D], W_edge[D:2 * D]
    W_ee, W_eg = W_edge[2 * D:2 * D + DE], W_edge[2 * D + DE:]
    wcat = jnp.concatenate([W_s, W_es, W_er], axis=1)

    z, gev, gn, tg = _pre(nodes, wcat, globals_, W_eg, W_g4, W_g,
                          b_g.reshape(1, DG))
    qrt = z[:, ZW - DE:]

    senders = senders.astype(I32)
    receivers = receivers.astype(I32)
    acc, u = _sc_run(z, qrt, senders, receivers, edge_weights, edges,
                     gev.reshape(DE))

    new_edges = _edge_post(u, edge_weights.reshape(E, 1), edges, W_ee,
                           b_edge.reshape(1, DE))
    new_nodes, new_global = _node_post(
        acc, nodes, W_e, W_r, gn, b_node.reshape(1, D), tg,
        W_gn, b_gn.reshape(1, DG), W_ge, b_ge.reshape(1, DG),
        W_fg, b_fg.reshape(1, DG))
    return (new_nodes, new_edges, new_global)
